# baseline (speedup 1.0000x reference)
import functools

import jax
import jax.numpy as jnp
import numpy as np
from jax.experimental import pallas as pl
from jax.experimental.pallas import tpu as pltpu

_N = 8192
_CHUNK = 256
_NCHUNK = _N // _CHUNK
_BLOCK_ROWS = 256


def _split(v):
    hi = v.astype(jnp.bfloat16)
    lo = (v - hi.astype(jnp.float32)).astype(jnp.bfloat16)
    return hi, lo


def _scan_kernel(x_ref, tri_ref, bd_ref, stri_ref, e_ref, o_ref):
    xh = x_ref[...].astype(jnp.bfloat16)
    tri = tri_ref[...]
    bd = bd_ref[...]

    totals = jnp.dot(xh, bd, preferred_element_type=jnp.float32)

    th, tl = _split(totals)
    stri = stri_ref[...]
    carries = jnp.dot(th, stri, preferred_element_type=jnp.float32) + jnp.dot(
        tl, stri, preferred_element_type=jnp.float32
    )
    ch, cl = _split(carries)

    for c in range(_NCHUNK):
        s = slice(c * _CHUNK, (c + 1) * _CHUNK)
        ec = e_ref[:, s]
        o_ref[:, s] = (
            jnp.dot(xh[:, s], tri, preferred_element_type=jnp.float32)
            + jnp.dot(ch, ec, preferred_element_type=jnp.float32)
            + jnp.dot(cl, ec, preferred_element_type=jnp.float32)
        )


@functools.partial(jax.jit, static_argnums=())
def _run(x, tri, bd, stri, e):
    m, n = x.shape
    grid = (m // _BLOCK_ROWS,)
    const = lambda shape: pl.BlockSpec(shape, lambda i: (0, 0))
    return pl.pallas_call(
        _scan_kernel,
        grid=grid,
        in_specs=[
            pl.BlockSpec((_BLOCK_ROWS, n), lambda i: (i, 0)),
            const((_CHUNK, _CHUNK)),
            const((_N, _NCHUNK)),
            const((_NCHUNK, _NCHUNK)),
            const((_NCHUNK, _N)),
        ],
        out_specs=pl.BlockSpec((_BLOCK_ROWS, n), lambda i: (i, 0)),
        out_shape=jax.ShapeDtypeStruct((m, n), x.dtype),
        compiler_params=pltpu.CompilerParams(
            dimension_semantics=("parallel",),
        ),
    )(x, tri, bd, stri, e)


def kernel(x):
    ii, jj = np.indices((_CHUNK, _CHUNK))
    tri = jnp.asarray((ii <= jj), dtype=jnp.bfloat16)
    bi, bj = np.indices((_N, _NCHUNK))
    bd = jnp.asarray((bi // _CHUNK == bj), dtype=jnp.bfloat16)
    ci, cj = np.indices((_NCHUNK, _NCHUNK))
    stri = jnp.asarray((ci < cj), dtype=jnp.bfloat16)
    ri, rj = np.indices((_NCHUNK, _N))
    e = jnp.asarray((rj // _CHUNK == ri), dtype=jnp.bfloat16)
    return _run(x, tri, bd, stri, e)
